# initial kernel scaffold (unmeasured)
import jax
import jax.numpy as jnp
from jax import lax
from jax.experimental import pallas as pl
from jax.experimental.pallas import tpu as pltpu

T = 4096
V_SHARD = 8192
D = 2048
HALF = T // 2
K = 8
CH = HALF // K


def kernel(ids, E):
    y = lax.axis_index("y")
    ids_local = ids - y * V_SHARD
    mask = (ids_local >= 0) & (ids_local < V_SHARD)
    idx = jnp.where(mask, ids_local, 0)
    p = jnp.where(mask[:, None], E[idx], 0.0).astype(jnp.bfloat16)

    def body(p_ref, out_ref, recv1, sbuf, recv2, s1, r1, s2, r2):
        x = lax.axis_index("x")
        yy = lax.axis_index("y")
        base = x * HALF

        barrier_sem = pltpu.get_barrier_semaphore()
        pl.semaphore_signal(barrier_sem, inc=1, device_id=(x, 1 - yy),
                            device_id_type=pl.DeviceIdType.MESH)
        pl.semaphore_signal(barrier_sem, inc=1, device_id=(1 - x, yy),
                            device_id_type=pl.DeviceIdType.MESH)
        pl.semaphore_wait(barrier_sem, 2)

        rdma1 = []
        for c in range(K):
            d = pltpu.make_async_remote_copy(
                src_ref=p_ref.at[pl.ds(base + c * CH, CH), :],
                dst_ref=recv1.at[pl.ds(c * CH, CH), :],
                send_sem=s1.at[c],
                recv_sem=r1.at[c],
                device_id=(x, 1 - yy),
                device_id_type=pl.DeviceIdType.MESH,
            )
            d.start()
            rdma1.append(d)

        rdma2 = []
        for c in range(K):
            rdma1[c].wait_recv()
            s = (p_ref[pl.ds(base + c * CH, CH), :]
                 + recv1[pl.ds(c * CH, CH), :])
            sbuf[pl.ds(c * CH, CH), :] = s
            d2 = pltpu.make_async_remote_copy(
                src_ref=sbuf.at[pl.ds(c * CH, CH), :],
                dst_ref=recv2.at[pl.ds(c * CH, CH), :],
                send_sem=s2.at[c],
                recv_sem=r2.at[c],
                device_id=(1 - x, yy),
                device_id_type=pl.DeviceIdType.MESH,
            )
            d2.start()
            rdma2.append(d2)
            out_ref[pl.ds(base + c * CH, CH), :] = s.astype(jnp.float32)

        obase = (1 - x) * HALF
        for c in range(K):
            rdma2[c].wait_recv()
            out_ref[pl.ds(obase + c * CH, CH), :] = (
                recv2[pl.ds(c * CH, CH), :].astype(jnp.float32))

        for c in range(K):
            rdma1[c].wait_send()
            rdma2[c].wait_send()

    return pl.pallas_call(
        body,
        out_shape=jax.ShapeDtypeStruct((T, D), jnp.float32),
        in_specs=[pl.BlockSpec(memory_space=pltpu.VMEM)],
        out_specs=pl.BlockSpec(memory_space=pltpu.VMEM),
        scratch_shapes=[
            pltpu.VMEM((HALF, D), jnp.bfloat16),
            pltpu.VMEM((HALF, D), jnp.bfloat16),
            pltpu.VMEM((HALF, D), jnp.bfloat16),
            pltpu.SemaphoreType.DMA((K,)),
            pltpu.SemaphoreType.DMA((K,)),
            pltpu.SemaphoreType.DMA((K,)),
            pltpu.SemaphoreType.DMA((K,)),
        ],
        compiler_params=pltpu.CompilerParams(collective_id=0),
    )(p)


# baseline (device time: 285677 ns/iter reference)
import jax
import jax.numpy as jnp
from jax import lax
from jax.experimental import pallas as pl
from jax.experimental.pallas import tpu as pltpu

T = 4096
V_SHARD = 8192
D = 2048
HALF = T // 2
K = 8
CH = HALF // K


def kernel(ids, E):
    y = lax.axis_index("y")
    ids_local = ids - y * V_SHARD
    mask = (ids_local >= 0) & (ids_local < V_SHARD)
    idx = jnp.where(mask, ids_local, 0)
    p = jnp.where(mask[:, None], E[idx], 0.0).astype(jnp.bfloat16)

    def body(p_ref, out_ref, recv1, s1, r1, s2, r2):
        x = lax.axis_index("x")
        yy = lax.axis_index("y")
        base = x * HALF

        barrier_sem = pltpu.get_barrier_semaphore()
        pl.semaphore_signal(barrier_sem, inc=1, device_id=(x, 1 - yy),
                            device_id_type=pl.DeviceIdType.MESH)
        pl.semaphore_signal(barrier_sem, inc=1, device_id=(1 - x, yy),
                            device_id_type=pl.DeviceIdType.MESH)
        pl.semaphore_wait(barrier_sem, 2)

        rdma1 = []
        for c in range(K):
            d = pltpu.make_async_remote_copy(
                src_ref=p_ref.at[pl.ds(base + c * CH, CH), :],
                dst_ref=recv1.at[pl.ds(c * CH, CH), :],
                send_sem=s1.at[c],
                recv_sem=r1.at[c],
                device_id=(x, 1 - yy),
                device_id_type=pl.DeviceIdType.MESH,
            )
            d.start()
            rdma1.append(d)

        rdma2 = []
        for c in range(K):
            rdma1[c].wait_recv()
            out_ref[pl.ds(base + c * CH, CH), :] = (
                p_ref[pl.ds(base + c * CH, CH), :]
                + recv1[pl.ds(c * CH, CH), :])
            d2 = pltpu.make_async_remote_copy(
                src_ref=out_ref.at[pl.ds(base + c * CH, CH), :],
                dst_ref=out_ref.at[pl.ds(base + c * CH, CH), :],
                send_sem=s2.at[c],
                recv_sem=r2.at[c],
                device_id=(1 - x, yy),
                device_id_type=pl.DeviceIdType.MESH,
            )
            d2.start()
            rdma2.append(d2)

        for c in range(K):
            rdma2[c].wait_recv()
        for c in range(K):
            rdma1[c].wait_send()
            rdma2[c].wait_send()

    out = pl.pallas_call(
        body,
        out_shape=jax.ShapeDtypeStruct((T, D), jnp.bfloat16),
        in_specs=[pl.BlockSpec(memory_space=pltpu.VMEM)],
        out_specs=pl.BlockSpec(memory_space=pltpu.VMEM),
        scratch_shapes=[
            pltpu.VMEM((HALF, D), jnp.bfloat16),
            pltpu.SemaphoreType.DMA((K,)),
            pltpu.SemaphoreType.DMA((K,)),
            pltpu.SemaphoreType.DMA((K,)),
            pltpu.SemaphoreType.DMA((K,)),
        ],
        compiler_params=pltpu.CompilerParams(collective_id=0),
    )(p)
    return out.astype(jnp.float32)


# device time: 192441 ns/iter; 1.4845x vs baseline; 1.4845x over previous
import jax
import jax.numpy as jnp
from jax import lax
from jax.experimental import pallas as pl
from jax.experimental.pallas import tpu as pltpu

T = 4096
V_SHARD = 8192
D = 2048
HALF = T // 2
K = 8
CH = HALF // K
NSEM = 32
WAVES = CH // NSEM


def kernel(ids, E):
    ids2d = ids[:, None]

    def body(ids_s, idv_ref, e_ref, out_ref, gstage, recv1, gsem, s1, r1, s2, r2):
        x = lax.axis_index("x")
        yy = lax.axis_index("y")
        base = x * HALF
        vlo = yy * V_SHARD

        barrier_sem = pltpu.get_barrier_semaphore()
        pl.semaphore_signal(barrier_sem, inc=1, device_id=(x, 1 - yy),
                            device_id_type=pl.DeviceIdType.MESH)
        pl.semaphore_signal(barrier_sem, inc=1, device_id=(1 - x, yy),
                            device_id_type=pl.DeviceIdType.MESH)
        pl.semaphore_wait(barrier_sem, 2)

        def row_copy(lidx, dst_row, k):
            return pltpu.make_async_copy(
                e_ref.at[pl.ds(lidx, 1), :],
                gstage.at[pl.ds(dst_row, 1), :],
                gsem.at[k],
            )

        def issue_row(t, dst_row, k):
            lidx = jnp.clip(ids_s[t] - vlo, 0, V_SHARD - 1)
            row_copy(lidx, dst_row, k).start()

        rdma1 = []
        rdma2 = []
        for c in range(K):
            lo = base + c * CH
            for k in range(NSEM):
                issue_row(lo + k, k, k)

            def wave(j, _, lo=lo):
                for k in range(NSEM):
                    row_copy(0, 0, k).wait()
                    issue_row(lo + j * NSEM + k, j * NSEM + k, k)
                return 0

            lax.fori_loop(1, WAVES, wave, 0)
            for k in range(NSEM):
                row_copy(0, 0, k).wait()

            out_ref[pl.ds(lo, CH), :] = gstage[:, :].astype(jnp.bfloat16)
            d1 = pltpu.make_async_remote_copy(
                src_ref=out_ref.at[pl.ds(lo, CH), :],
                dst_ref=recv1.at[pl.ds(c * CH, CH), :],
                send_sem=s1.at[c],
                recv_sem=r1.at[c],
                device_id=(x, 1 - yy),
                device_id_type=pl.DeviceIdType.MESH,
            )
            d1.start()
            rdma1.append(d1)

        for c in range(K):
            lo = base + c * CH
            rdma1[c].wait_recv()
            rdma1[c].wait_send()
            sel = (idv_ref[pl.ds(lo, CH), :] >= vlo) & (
                idv_ref[pl.ds(lo, CH), :] < vlo + V_SHARD)
            out_ref[pl.ds(lo, CH), :] = jnp.where(
                sel, out_ref[pl.ds(lo, CH), :], recv1[pl.ds(c * CH, CH), :])
            d2 = pltpu.make_async_remote_copy(
                src_ref=out_ref.at[pl.ds(lo, CH), :],
                dst_ref=out_ref.at[pl.ds(lo, CH), :],
                send_sem=s2.at[c],
                recv_sem=r2.at[c],
                device_id=(1 - x, yy),
                device_id_type=pl.DeviceIdType.MESH,
            )
            d2.start()
            rdma2.append(d2)

        for c in range(K):
            rdma2[c].wait_recv()
        for c in range(K):
            rdma2[c].wait_send()

    out = pl.pallas_call(
        body,
        out_shape=jax.ShapeDtypeStruct((T, D), jnp.bfloat16),
        in_specs=[
            pl.BlockSpec(memory_space=pltpu.SMEM),
            pl.BlockSpec(memory_space=pltpu.VMEM),
            pl.BlockSpec(memory_space=pl.ANY),
        ],
        out_specs=pl.BlockSpec(memory_space=pltpu.VMEM),
        scratch_shapes=[
            pltpu.VMEM((CH, D), jnp.float32),
            pltpu.VMEM((HALF, D), jnp.bfloat16),
            pltpu.SemaphoreType.DMA((NSEM,)),
            pltpu.SemaphoreType.DMA((K,)),
            pltpu.SemaphoreType.DMA((K,)),
            pltpu.SemaphoreType.DMA((K,)),
            pltpu.SemaphoreType.DMA((K,)),
        ],
        compiler_params=pltpu.CompilerParams(collective_id=0),
    )(ids, ids2d, E)
    return out.astype(jnp.float32)
